# Initial kernel scaffold; baseline (speedup 1.0000x reference)
import jax
import jax.numpy as jnp
from jax.experimental import pallas as pl


def kernel(e_in, e_out, in_mem, edge_index, edge_weight):
    raise NotImplementedError("write your pallas kernel here")


# speedup vs baseline: 6.3204x; 6.3204x over previous
import functools

import jax
import jax.numpy as jnp
from jax import lax
from jax.experimental import pallas as pl
from jax.experimental.pallas import tpu as pltpu
from jax.experimental.pallas import tpu_sc as plsc

N = 10000
USER_NUM = 5000
D = 128
E = 320000
ALPHA = 0.1

NC = 2
NS = 16
NW = NC * NS
K = 128
CHUNKS = -(-E // (NW * K))
EPW = CHUNKS * K
E_PAD = EPW * NW
ROWS_PER_TILE = 8 * (-(-N // (8 * NS)))
N_PAD = ROWS_PER_TILE * NS


def _sc_spmm(e_out, src, dst, w, zrows):
    mesh = plsc.VectorSubcoreMesh(core_axis_name="c", subcore_axis_name="s")

    @functools.partial(
        pl.kernel,
        mesh=mesh,
        out_type=jax.ShapeDtypeStruct((NC * N_PAD, D), jnp.float32),
        scratch_types=[
            pltpu.VMEM((K,), jnp.int32),
            pltpu.VMEM((K,), jnp.int32),
            pltpu.VMEM((K,), jnp.float32),
            pltpu.VMEM((K, D), jnp.float32),
            pltpu.VMEM_SHARED((N_PAD, D), jnp.float32),
            pltpu.SemaphoreType.DMA,
        ],
    )
    def k(eout_hbm, src_hbm, dst_hbm, w_hbm, z_hbm, out_hbm,
          src_v, dst_v, w_v, rows_v, acc, sem):
        c = lax.axis_index("c")
        s = lax.axis_index("s")
        wid = c * NS + s

        pltpu.sync_copy(z_hbm, acc.at[pl.ds(s * ROWS_PER_TILE, ROWS_PER_TILE)])
        plsc.subcore_barrier()

        def chunk_body(i, carry):
            base = wid * EPW + i * K
            pltpu.sync_copy(src_hbm.at[pl.ds(base, K)], src_v)
            pltpu.sync_copy(dst_hbm.at[pl.ds(base, K)], dst_v)
            pltpu.sync_copy(w_hbm.at[pl.ds(base, K)], w_v)
            pltpu.async_copy(eout_hbm.at[src_v], rows_v, sem).wait()

            def group_body(g, rc):
                wvec = w_v[pl.ds(g * 16, 16)]
                for i in range(16):
                    r = g * 16 + i
                    wi = wvec[i]
                    for j in range(D // 16):
                        sl = pl.ds(j * 16, 16)
                        rows_v[r, sl] = rows_v[r, sl] * wi
                return rc

            lax.fori_loop(0, K // 16, group_body, 0)
            pltpu.sync_copy(rows_v, acc.at[dst_v], add=True)
            return carry

        lax.fori_loop(0, CHUNKS, chunk_body, 0)
        plsc.subcore_barrier()

        pltpu.sync_copy(
            acc.at[pl.ds(s * ROWS_PER_TILE, ROWS_PER_TILE)],
            out_hbm.at[pl.ds(c * N_PAD + s * ROWS_PER_TILE, ROWS_PER_TILE)],
        )

    return k(e_out, src, dst, w, zrows)


def _tc_combine(p0, p1, e_in):
    R = 1000

    def body(p0_ref, p1_ref, ein_ref, o_ref):
        o_ref[:] = (1.0 - ALPHA) * (p0_ref[:] + p1_ref[:]) + ALPHA * ein_ref[:]

    spec = pl.BlockSpec((R, D), lambda i: (i, 0))
    return pl.pallas_call(
        body,
        grid=(N // R,),
        in_specs=[spec, spec, spec],
        out_specs=spec,
        out_shape=jax.ShapeDtypeStruct((N, D), jnp.float32),
    )(p0, p1, e_in)


def kernel(e_in, e_out, in_mem, edge_index, edge_weight):
    del in_mem
    dst = edge_index[0].astype(jnp.int32)
    src = edge_index[1].astype(jnp.int32)
    w = edge_weight.astype(jnp.float32)
    pad = E_PAD - E
    src = jnp.concatenate([src, jnp.zeros((pad,), jnp.int32)])
    dst = jnp.concatenate([dst, jnp.zeros((pad,), jnp.int32)])
    w = jnp.concatenate([w, jnp.zeros((pad,), jnp.float32)])
    zrows = jnp.zeros((ROWS_PER_TILE, D), jnp.float32)

    partials = _sc_spmm(e_out, src, dst, w, zrows)
    out = _tc_combine(partials[:N], partials[N_PAD:N_PAD + N], e_in)
    return (out[:USER_NUM], out[USER_NUM:])


# speedup vs baseline: 6.4305x; 1.0174x over previous
import functools

import jax
import jax.numpy as jnp
from jax import lax
from jax.experimental import pallas as pl
from jax.experimental.pallas import tpu as pltpu
from jax.experimental.pallas import tpu_sc as plsc

N = 10000
USER_NUM = 5000
D = 128
E = 320000
ALPHA = 0.1

NC = 2
NS = 16
NW = NC * NS
K = 128
NBUF = 2
CHUNKS = NBUF * (-(-E // (NW * K * NBUF)))
EPW = CHUNKS * K
E_PAD = EPW * NW
ROWS_PER_TILE = 8 * (-(-N // (8 * NS)))
N_PAD = ROWS_PER_TILE * NS


def _sc_spmm(e_out, src, dst, w, zrows):
    mesh = plsc.VectorSubcoreMesh(core_axis_name="c", subcore_axis_name="s")

    @functools.partial(
        pl.kernel,
        mesh=mesh,
        out_type=jax.ShapeDtypeStruct((NC * N_PAD, D), jnp.float32),
        scratch_types=[
            pltpu.VMEM((NBUF, K), jnp.int32),
            pltpu.VMEM((NBUF, K), jnp.int32),
            pltpu.VMEM((NBUF, K), jnp.float32),
            pltpu.VMEM((NBUF, K, D), jnp.float32),
            pltpu.VMEM_SHARED((N_PAD, D), jnp.float32),
            pltpu.SemaphoreType.DMA((NBUF,)),
            pltpu.SemaphoreType.DMA((NBUF,)),
        ],
    )
    def k(eout_hbm, src_hbm, dst_hbm, w_hbm, z_hbm, out_hbm,
          src_v, dst_v, w_v, rows_v, acc, gsem, msem):
        c = lax.axis_index("c")
        s = lax.axis_index("s")
        wid = c * NS + s

        def meta_copies(j, b):
            base = wid * EPW + j * K
            return [
                pltpu.make_async_copy(
                    src_hbm.at[pl.ds(base, K)], src_v.at[b], msem.at[b]),
                pltpu.make_async_copy(
                    dst_hbm.at[pl.ds(base, K)], dst_v.at[b], msem.at[b]),
                pltpu.make_async_copy(
                    w_hbm.at[pl.ds(base, K)], w_v.at[b], msem.at[b]),
            ]

        def gather_copy(b):
            return pltpu.make_async_copy(
                eout_hbm.at[src_v.at[b]], rows_v.at[b], gsem.at[b])

        pltpu.sync_copy(z_hbm, acc.at[pl.ds(s * ROWS_PER_TILE, ROWS_PER_TILE)])
        plsc.subcore_barrier()

        for d in meta_copies(0, 0):
            d.start()
        for d in meta_copies(1, 1):
            d.start()
        for d in meta_copies(0, 0):
            d.wait()
        gather_copy(0).start()

        def round_body(rnd, carry):
            for b in range(NBUF):
                i = rnd * NBUF + b
                nb = (b + 1) % NBUF

                @pl.when(i + 1 < CHUNKS)
                def _():
                    for d in meta_copies(i + 1, nb):
                        d.wait()
                    gather_copy(nb).start()

                gather_copy(b).wait()
                rows_b = rows_v.at[b]

                def group_body(g, rc):
                    wvec = w_v[b, pl.ds(g * 16, 16)]
                    for u in range(16):
                        r = g * 16 + u
                        wu = wvec[u]
                        for j in range(D // 16):
                            sl = pl.ds(j * 16, 16)
                            rows_b[r, sl] = rows_b[r, sl] * wu
                    return rc

                lax.fori_loop(0, K // 16, group_body, 0)
                pltpu.sync_copy(rows_b, acc.at[dst_v.at[b]], add=True)

                @pl.when(i + NBUF < CHUNKS)
                def _():
                    for d in meta_copies(i + NBUF, b):
                        d.start()

            return carry

        lax.fori_loop(0, CHUNKS // NBUF, round_body, 0)
        plsc.subcore_barrier()

        pltpu.sync_copy(
            acc.at[pl.ds(s * ROWS_PER_TILE, ROWS_PER_TILE)],
            out_hbm.at[pl.ds(c * N_PAD + s * ROWS_PER_TILE, ROWS_PER_TILE)],
        )

    return k(e_out, src, dst, w, zrows)


def _tc_combine(p0, p1, e_in):
    R = 1000

    def body(p0_ref, p1_ref, ein_ref, o_ref):
        o_ref[:] = (1.0 - ALPHA) * (p0_ref[:] + p1_ref[:]) + ALPHA * ein_ref[:]

    spec = pl.BlockSpec((R, D), lambda i: (i, 0))
    return pl.pallas_call(
        body,
        grid=(N // R,),
        in_specs=[spec, spec, spec],
        out_specs=spec,
        out_shape=jax.ShapeDtypeStruct((N, D), jnp.float32),
    )(p0, p1, e_in)


def kernel(e_in, e_out, in_mem, edge_index, edge_weight):
    del in_mem
    dst = edge_index[0].astype(jnp.int32)
    src = edge_index[1].astype(jnp.int32)
    w = edge_weight.astype(jnp.float32)
    pad = E_PAD - E
    src = jnp.concatenate([src, jnp.zeros((pad,), jnp.int32)])
    dst = jnp.concatenate([dst, jnp.zeros((pad,), jnp.int32)])
    w = jnp.concatenate([w, jnp.zeros((pad,), jnp.float32)])
    zrows = jnp.zeros((ROWS_PER_TILE, D), jnp.float32)

    partials = _sc_spmm(e_out, src, dst, w, zrows)
    out = _tc_combine(partials[:N], partials[N_PAD:N_PAD + N], e_in)
    return (out[:USER_NUM], out[USER_NUM:])
